# Initial kernel scaffold; baseline (speedup 1.0000x reference)
import jax
import jax.numpy as jnp
from jax.experimental import pallas as pl


def kernel(unit_indicator, type, direction, resource, repeat, amount, n, type_table, dir_table, res_table, rep_table, conv1_w, conv1_b, conv2_w, conv2_b):
    raise NotImplementedError("write your pallas kernel here")


# speedup vs baseline: 43.8634x; 43.8634x over previous
import jax
import jax.numpy as jnp
from jax import lax
from jax.experimental import pallas as pl
from jax.experimental.pallas import tpu as pltpu
from jax.experimental.pallas import tpu_sc as plsc

B, H, W, Q, E = 16, 48, 48, 20, 32
N = B * H * W
HW = H * W
NBINS = 20
NW = 32
CELLS_PER_W = N // NW
CHUNK = 384
NCHUNK = CELLS_PER_W // CHUNK
GROUPS = CHUNK // 16


def _sc_coef_body(type_h, dir_h, res_h, rep_h, amt_h, n_h, w1_h, coef_h,
                  type_v, dir_v, res_v, rep_v, amt_v, n_v, w1_v, hist_v):
    wid = lax.axis_index("c") * 16 + lax.axis_index("s")
    lane = lax.iota(jnp.int32, 16)
    lane_q = lane * Q
    c18 = jnp.full((16,), 18, jnp.int32)
    c19 = jnp.full((16,), 19, jnp.int32)
    zeros16 = jnp.zeros((16,), jnp.float32)
    pltpu.sync_copy(w1_h, w1_v)
    w1qs = [plsc.load_gather(w1_v, [jnp.full((16,), q + 1, jnp.int32)])
            for q in range(Q)]
    for chunk in range(NCHUNK):
        cell_base = wid * CELLS_PER_W + chunk * CHUNK
        eb = cell_base * Q
        pltpu.sync_copy(type_h.at[pl.ds(eb, CHUNK * Q)], type_v)
        pltpu.sync_copy(dir_h.at[pl.ds(eb, CHUNK * Q)], dir_v)
        pltpu.sync_copy(res_h.at[pl.ds(eb, CHUNK * Q)], res_v)
        pltpu.sync_copy(rep_h.at[pl.ds(eb, CHUNK * Q)], rep_v)
        pltpu.sync_copy(amt_h.at[pl.ds(eb, CHUNK * Q)], amt_v)
        pltpu.sync_copy(n_h.at[pl.ds(eb, CHUNK * Q)], n_v)

        def zero_body(i, _):
            for r in range(NBINS):
                hist_v[r, pl.ds(i * 16, 16)] = zeros16
            return 0
        lax.fori_loop(0, GROUPS, zero_body, 0)

        def g_body(g, _):
            cell_vec = g * 16 + lane
            idx0 = g * (16 * Q) + lane_q
            for q in range(Q):
                idx = idx0 + q
                w1q = w1qs[q]
                tv = plsc.load_gather(type_v, [idx])
                plsc.addupdate_scatter(hist_v, [tv, cell_vec], w1q)
                dv = plsc.load_gather(dir_v, [idx])
                plsc.addupdate_scatter(hist_v, [dv + 6, cell_vec], w1q)
                rv = plsc.load_gather(res_v, [idx])
                plsc.addupdate_scatter(hist_v, [rv + 11, cell_vec], w1q)
                pv = plsc.load_gather(rep_v, [idx])
                plsc.addupdate_scatter(hist_v, [16 + (pv != 0).astype(jnp.int32),
                                                cell_vec], w1q)
                av = plsc.load_gather(amt_v, [idx])
                plsc.addupdate_scatter(hist_v, [c18, cell_vec], av * w1q)
                nv = plsc.load_gather(n_v, [idx])
                plsc.addupdate_scatter(hist_v, [c19, cell_vec], nv * w1q)
            return 0
        lax.fori_loop(0, GROUPS, g_body, 0)

        pltpu.sync_copy(hist_v, coef_h.at[:, pl.ds(cell_base, CHUNK)])


def _bf(x):
    return x.astype(jnp.bfloat16).astype(jnp.float32)


def _tc_body(ms_ref, coef_ref, mask_ref, par_ref, out_ref):
    b1 = par_ref[0]
    b2 = par_ref[1]
    hall = lax.dot_general(ms_ref[...], coef_ref[...], (((1,), (0,)), ((), ())),
                           preferred_element_type=jnp.float32,
                           precision=lax.Precision.HIGHEST)
    h = _bf(hall[0:E, :] + b1) * par_ref[2]
    h = h + _bf(hall[E:2 * E, :] + b1) * par_ref[3]
    h = h + _bf(hall[2 * E:3 * E, :] + b1) * par_ref[4]
    h = h + _bf(coef_ref[18:19, :] + b1) * par_ref[5]
    h = h + _bf(hall[3 * E:4 * E, :] + b1) * par_ref[6]
    h = h + _bf(coef_ref[19:20, :] + b1) * par_ref[7]
    h = h + b2
    h = jnp.where(h >= 0, h, 0.01 * h)
    out_ref[0] = h * mask_ref[0]


def kernel(unit_indicator, type, direction, resource, repeat, amount, n,
           type_table, dir_table, res_table, rep_table,
           conv1_w, conv1_b, conv2_w, conv2_b):
    type_f = type.reshape(-1)
    dir_f = direction.reshape(-1)
    res_f = resource.reshape(-1)
    rep_f = repeat.reshape(-1)
    amt_f = _bf(amount).reshape(-1)
    n_f = _bf(n).reshape(-1)
    w1_pad = jnp.pad(_bf(conv1_w), (1, 11))

    mesh = plsc.VectorSubcoreMesh(core_axis_name="c", subcore_axis_name="s")
    coef = pl.kernel(
        _sc_coef_body,
        out_type=jax.ShapeDtypeStruct((NBINS, N), jnp.float32),
        mesh=mesh,
        compiler_params=pltpu.CompilerParams(needs_layout_passes=False),
        scratch_types=[
            pltpu.VMEM((CHUNK * Q,), jnp.int32),
            pltpu.VMEM((CHUNK * Q,), jnp.int32),
            pltpu.VMEM((CHUNK * Q,), jnp.int32),
            pltpu.VMEM((CHUNK * Q,), jnp.int32),
            pltpu.VMEM((CHUNK * Q,), jnp.float32),
            pltpu.VMEM((CHUNK * Q,), jnp.float32),
            pltpu.VMEM((32,), jnp.float32),
            pltpu.VMEM((NBINS, CHUNK), jnp.float32),
        ],
    )(type_f, dir_f, res_f, rep_f, amt_f, n_f, w1_pad)

    ms = jnp.zeros((4 * E, NBINS), jnp.float32)
    ms = ms.at[0:E, 0:6].set(_bf(type_table).T)
    ms = ms.at[E:2 * E, 6:11].set(_bf(dir_table).T)
    ms = ms.at[2 * E:3 * E, 11:16].set(_bf(res_table).T)
    ms = ms.at[3 * E:4 * E, 16:18].set(_bf(rep_table).T)
    w2b = _bf(conv2_w)
    par = jnp.stack([conv1_b, conv2_b, w2b[0], w2b[1], w2b[2], w2b[3],
                     w2b[4], w2b[5]])
    mask_f = unit_indicator.reshape(B, 1, HW).astype(jnp.float32)

    out = pl.pallas_call(
        _tc_body,
        grid=(B,),
        in_specs=[
            pl.BlockSpec((4 * E, NBINS), lambda b: (0, 0)),
            pl.BlockSpec((NBINS, HW), lambda b: (0, b)),
            pl.BlockSpec((1, 1, HW), lambda b: (b, 0, 0)),
            pl.BlockSpec(memory_space=pltpu.SMEM),
        ],
        out_specs=pl.BlockSpec((1, E, HW), lambda b: (b, 0, 0)),
        out_shape=jax.ShapeDtypeStruct((B, E, HW), jnp.float32),
    )(ms, coef, mask_f, par)
    return out.reshape(B, E, H, W)
